# Initial kernel scaffold; baseline (speedup 1.0000x reference)
import jax
import jax.numpy as jnp
from jax.experimental import pallas as pl


def kernel(metadata, boxes, p2, p3, p4, p5):
    raise NotImplementedError("write your pallas kernel here")


# speedup vs baseline: 3.5198x; 3.5198x over previous
import functools

import numpy as np
import jax
import jax.numpy as jnp
from jax import lax
from jax.experimental import pallas as pl
from jax.experimental.pallas import tpu as pltpu
from jax.experimental.pallas import tpu_sc as plsc

NB = 1000
NB_PAD = 1024
PH = PW = 7
NS = PH * PW
NE = 4 * NS
NE_PAD = 208
NW_PAD = 112
CHUNK = 104
SAMP_A = CHUNK // 4
NUM_WORKERS = 32
BOX_PER_W = NB_PAD // NUM_WORKERS

_FY = [float(np.float32(i) / np.float32(6.0)) for i in range(7)]


def _prelude_body(sizes, bases, meta_ref, bx_ref, idx_ref, w_ref):
    rows = meta_ref[0, 0]
    cols = meta_ref[0, 1]
    x1 = bx_ref[0, :]
    y1 = bx_ref[1, :]
    x2 = bx_ref[2, :]
    y2 = bx_ref[3, :]
    h = y2 - y1
    w = x2 - x1
    image_area = rows * cols
    rl = jnp.log(jnp.sqrt(h * w) / jnp.sqrt(image_area)) / jnp.log(2.0)
    rl = jnp.minimum(5.0, jnp.maximum(2.0, 4.0 + jnp.round(rl)))
    lvl = (rl - 2.0).astype(jnp.int32)

    Si = jnp.full(lvl.shape, sizes[0], jnp.int32)
    base = jnp.full(lvl.shape, bases[0], jnp.int32)
    for L in range(1, 4):
        Si = jnp.where(lvl == L, jnp.int32(sizes[L]), Si)
        base = jnp.where(lvl == L, jnp.int32(bases[L]), base)
    Sf = Si.astype(jnp.float32)
    Hm1 = Sf - 1.0

    y1n = y1 / rows
    x1n = x1 / cols
    y2n = y2 / rows
    x2n = x2 / cols

    ygrid = []
    xgrid = []
    for i in range(7):
        ys = (y1n + _FY[i] * (y2n - y1n)) * Hm1
        y0f = jnp.clip(jnp.floor(ys), 0.0, Hm1)
        y0 = y0f.astype(jnp.int32)
        y1c = jnp.clip(y0 + 1, 0, Si - 1)
        wy = jnp.clip(ys - y0f, 0.0, 1.0)
        ygrid.append((y0, y1c, wy))
        xs = (x1n + _FY[i] * (x2n - x1n)) * Hm1
        x0f = jnp.clip(jnp.floor(xs), 0.0, Hm1)
        x0 = x0f.astype(jnp.int32)
        x1c = jnp.clip(x0 + 1, 0, Si - 1)
        wx = jnp.clip(xs - x0f, 0.0, 1.0)
        xgrid.append((x0, x1c, wx))

    for i in range(7):
        y0, y1c, wy = ygrid[i]
        top = base + y0 * Si
        bot = base + y1c * Si
        for j in range(7):
            x0, x1c, wx = xgrid[j]
            s = i * 7 + j
            idx_ref[4 * s + 0, :] = top + x0
            idx_ref[4 * s + 1, :] = top + x1c
            idx_ref[4 * s + 2, :] = bot + x0
            idx_ref[4 * s + 3, :] = bot + x1c
            w_ref[2 * s + 0, :] = wx
            w_ref[2 * s + 1, :] = wy
    idx_ref[pl.ds(NE, NE_PAD - NE), :] = jnp.zeros(
        (NE_PAD - NE, NB_PAD), jnp.int32)
    w_ref[pl.ds(2 * NS, NW_PAD - 2 * NS), :] = jnp.zeros(
        (NW_PAD - 2 * NS, NB_PAD), jnp.float32)


def _make_sc_kernel(table_rows, C):
    mesh = plsc.VectorSubcoreMesh(core_axis_name="c", subcore_axis_name="s")

    @functools.partial(
        pl.kernel,
        mesh=mesh,
        out_type=jax.ShapeDtypeStruct((NB * NS * C,), jnp.float32),
        scratch_types=[
            pltpu.VMEM((CHUNK,), jnp.int32),
            pltpu.VMEM((CHUNK,), jnp.int32),
            pltpu.VMEM((NW_PAD,), jnp.float32),
            pltpu.VMEM((CHUNK, C), jnp.float32),
            pltpu.VMEM((CHUNK, C), jnp.float32),
            pltpu.VMEM((NS * C,), jnp.float32),
            pltpu.SemaphoreType.DMA,
        ],
    )
    def sc_kernel(table, idx_all, w_all, out, idx_a, idx_b, w_v,
                  rows_a, rows_b, out_v, sem):
        wid = lax.axis_index("s") * 2 + lax.axis_index("c")

        def box_body(t, _):
            b = wid * BOX_PER_W + t
            pltpu.sync_copy(idx_all.at[pl.ds(b * NE_PAD, CHUNK)], idx_a)
            pltpu.sync_copy(idx_all.at[pl.ds(b * NE_PAD + CHUNK, CHUNK)], idx_b)
            pltpu.sync_copy(w_all.at[pl.ds(b * NW_PAD, NW_PAD)], w_v)
            cp1 = pltpu.async_copy(table.at[idx_a], rows_a, sem)
            cp2 = pltpu.async_copy(table.at[idx_b], rows_b, sem)
            cp1.wait()
            cp2.wait()

            def samp(s, rows_ref, off):
                e = 4 * s - off
                wv = w_v[pl.ds(2 * s, 16)]
                wx = jnp.full((16,), wv[0], jnp.float32)
                wy = jnp.full((16,), wv[1], jnp.float32)
                owx = 1.0 - wx
                owy = 1.0 - wy
                for k in range(C // 16):
                    sl = pl.ds(16 * k, 16)
                    va = rows_ref[e + 0, sl]
                    vb = rows_ref[e + 1, sl]
                    vc = rows_ref[e + 2, sl]
                    vd = rows_ref[e + 3, sl]
                    top = owx * va + wx * vb
                    bot = owx * vc + wx * vd
                    out_v[pl.ds(s * C + 16 * k, 16)] = owy * top + wy * bot

            def samp_a(s, carry):
                samp(s, rows_a, 0)
                return carry

            def samp_b(s, carry):
                samp(s, rows_b, CHUNK)
                return carry

            lax.fori_loop(0, SAMP_A, samp_a, 0)
            lax.fori_loop(SAMP_A, NS, samp_b, 0)

            @pl.when(b < NB)
            def _():
                pltpu.sync_copy(out_v, out.at[pl.ds(b * NS * C, NS * C)])

            return 0

        lax.fori_loop(0, BOX_PER_W, box_body, 0)

    return sc_kernel


def kernel(metadata, boxes, p2, p3, p4, p5):
    C = p2.shape[-1]
    levels = [p2, p3, p4, p5]
    sizes = [p.shape[1] for p in levels]
    counts = [p.shape[1] * p.shape[2] for p in levels]
    bases = [0, counts[0], counts[0] + counts[1], counts[0] + counts[1] + counts[2]]
    table_rows = sum(counts)

    table = jnp.concatenate([p.reshape(cnt, C) for p, cnt in zip(levels, counts)], axis=0)

    n = boxes.shape[1]
    bx = jnp.transpose(boxes[0], (1, 0))
    bx = jnp.pad(bx, ((0, 0), (0, NB_PAD - n)))

    idx_t, w_t = pl.pallas_call(
        functools.partial(_prelude_body, sizes, bases),
        out_shape=[
            jax.ShapeDtypeStruct((NE_PAD, NB_PAD), jnp.int32),
            jax.ShapeDtypeStruct((NW_PAD, NB_PAD), jnp.float32),
        ],
        in_specs=[
            pl.BlockSpec(memory_space=pltpu.SMEM),
            pl.BlockSpec(memory_space=pltpu.VMEM),
        ],
    )(metadata, bx)

    idx_all = jnp.transpose(idx_t, (1, 0)).reshape(-1)
    w_all = jnp.transpose(w_t, (1, 0)).reshape(-1)

    out = _make_sc_kernel(table_rows, C)(table, idx_all, w_all)
    return out.reshape(1, n, PH, PW, C)


# speedup vs baseline: 4.8995x; 1.3920x over previous
import functools

import numpy as np
import jax
import jax.numpy as jnp
from jax import lax
from jax.experimental import pallas as pl
from jax.experimental.pallas import tpu as pltpu
from jax.experimental.pallas import tpu_sc as plsc

NB = 1000
NB_PAD = 1024
PH = PW = 7
NS = PH * PW
NE = 4 * NS
NE_PAD = 208
NE_G = 200
NW_PAD = 112
CHUNK = 104
SAMP_A = CHUNK // 4
NUM_WORKERS = 32
BOX_PER_W = NB_PAD // NUM_WORKERS

_FY = [float(np.float32(i) / np.float32(6.0)) for i in range(7)]


def _prelude_body(sizes, bases, meta_ref, bx_ref, idx_ref, w_ref):
    rows = meta_ref[0, 0]
    cols = meta_ref[0, 1]
    x1 = bx_ref[0, :]
    y1 = bx_ref[1, :]
    x2 = bx_ref[2, :]
    y2 = bx_ref[3, :]
    h = y2 - y1
    w = x2 - x1
    image_area = rows * cols
    rl = jnp.log(jnp.sqrt(h * w) / jnp.sqrt(image_area)) / jnp.log(2.0)
    rl = jnp.minimum(5.0, jnp.maximum(2.0, 4.0 + jnp.round(rl)))
    lvl = (rl - 2.0).astype(jnp.int32)

    Si = jnp.full(lvl.shape, sizes[0], jnp.int32)
    base = jnp.full(lvl.shape, bases[0], jnp.int32)
    for L in range(1, 4):
        Si = jnp.where(lvl == L, jnp.int32(sizes[L]), Si)
        base = jnp.where(lvl == L, jnp.int32(bases[L]), base)
    Sf = Si.astype(jnp.float32)
    Hm1 = Sf - 1.0

    y1n = y1 / rows
    x1n = x1 / cols
    y2n = y2 / rows
    x2n = x2 / cols

    ygrid = []
    xgrid = []
    for i in range(7):
        ys = (y1n + _FY[i] * (y2n - y1n)) * Hm1
        y0f = jnp.clip(jnp.floor(ys), 0.0, Hm1)
        y0 = y0f.astype(jnp.int32)
        y1c = jnp.clip(y0 + 1, 0, Si - 1)
        wy = jnp.clip(ys - y0f, 0.0, 1.0)
        ygrid.append((y0, y1c, wy))
        xs = (x1n + _FY[i] * (x2n - x1n)) * Hm1
        x0f = jnp.clip(jnp.floor(xs), 0.0, Hm1)
        x0 = x0f.astype(jnp.int32)
        x1c = jnp.clip(x0 + 1, 0, Si - 1)
        wx = jnp.clip(xs - x0f, 0.0, 1.0)
        xgrid.append((x0, x1c, wx))

    for i in range(7):
        y0, y1c, wy = ygrid[i]
        top = base + y0 * Si
        bot = base + y1c * Si
        for j in range(7):
            x0, x1c, wx = xgrid[j]
            s = i * 7 + j
            idx_ref[4 * s + 0, :] = top + x0
            idx_ref[4 * s + 1, :] = top + x1c
            idx_ref[4 * s + 2, :] = bot + x0
            idx_ref[4 * s + 3, :] = bot + x1c
            w_ref[2 * s + 0, :] = wx
            w_ref[2 * s + 1, :] = wy
    idx_ref[pl.ds(NE, NE_PAD - NE), :] = jnp.zeros(
        (NE_PAD - NE, NB_PAD), jnp.int32)
    w_ref[pl.ds(2 * NS, NW_PAD - 2 * NS), :] = jnp.zeros(
        (NW_PAD - 2 * NS, NB_PAD), jnp.float32)


def _make_sc_kernel(table_rows, C):
    mesh = plsc.VectorSubcoreMesh(core_axis_name="c", subcore_axis_name="s")

    @functools.partial(
        pl.kernel,
        mesh=mesh,
        out_type=jax.ShapeDtypeStruct((NB * NS * C,), jnp.float32),
        scratch_types=[
            pltpu.VMEM((BOX_PER_W * NE_PAD,), jnp.int32),
            pltpu.VMEM((BOX_PER_W * NW_PAD,), jnp.float32),
            pltpu.VMEM((NE_G, C), jnp.float32),
            pltpu.VMEM((NE_G, C), jnp.float32),
            pltpu.VMEM((NS * C,), jnp.float32),
            pltpu.SemaphoreType.DMA,
            pltpu.SemaphoreType.DMA,
        ],
    )
    def sc_kernel(table, idx_all, w_all, out, idx_sub, w_sub,
                  rows0, rows1, out_v, sem0, sem1):
        wid = lax.axis_index("s") * 2 + lax.axis_index("c")

        pltpu.sync_copy(
            idx_all.at[pl.ds(wid * BOX_PER_W * NE_PAD, BOX_PER_W * NE_PAD)],
            idx_sub)
        pltpu.sync_copy(
            w_all.at[pl.ds(wid * BOX_PER_W * NW_PAD, BOX_PER_W * NW_PAD)],
            w_sub)

        def issue(t, buf, sem):
            o = t * NE_PAD
            pltpu.async_copy(table.at[idx_sub.at[pl.ds(o, CHUNK)]],
                             buf.at[pl.ds(0, CHUNK)], sem)
            pltpu.async_copy(table.at[idx_sub.at[pl.ds(o + CHUNK, NE_G - CHUNK)]],
                             buf.at[pl.ds(CHUNK, NE_G - CHUNK)], sem)

        def wait_buf(buf, sem):
            pltpu.make_async_copy(table.at[pl.ds(0, CHUNK)],
                                  buf.at[pl.ds(0, CHUNK)], sem).wait()
            pltpu.make_async_copy(table.at[pl.ds(0, NE_G - CHUNK)],
                                  buf.at[pl.ds(CHUNK, NE_G - CHUNK)], sem).wait()

        def compute(t, buf):
            def samp(s, carry):
                wv = w_sub[pl.ds(t * NW_PAD + 2 * s, 16)]
                wx = jnp.full((16,), wv[0], jnp.float32)
                wy = jnp.full((16,), wv[1], jnp.float32)
                owx = 1.0 - wx
                owy = 1.0 - wy
                e = 4 * s
                for k in range(C // 16):
                    sl = pl.ds(16 * k, 16)
                    va = buf[e + 0, sl]
                    vb = buf[e + 1, sl]
                    vc = buf[e + 2, sl]
                    vd = buf[e + 3, sl]
                    top = owx * va + wx * vb
                    bot = owx * vc + wx * vd
                    out_v[pl.ds(s * C + 16 * k, 16)] = owy * top + wy * bot
                return carry

            lax.fori_loop(0, NS, samp, 0)

        issue(0, rows0, sem0)
        issue(1, rows1, sem1)

        def pair_body(u, _):
            for par, (buf, sem) in enumerate(((rows0, sem0), (rows1, sem1))):
                t = 2 * u + par
                b = wid * BOX_PER_W + t
                wait_buf(buf, sem)
                compute(t, buf)

                @pl.when(b < NB)
                def _():
                    pltpu.sync_copy(out_v, out.at[pl.ds(b * NS * C, NS * C)])

                @pl.when(t + 2 < BOX_PER_W)
                def _():
                    issue(t + 2, buf, sem)

            return 0

        lax.fori_loop(0, BOX_PER_W // 2, pair_body, 0)

    return sc_kernel


def kernel(metadata, boxes, p2, p3, p4, p5):
    C = p2.shape[-1]
    levels = [p2, p3, p4, p5]
    sizes = [p.shape[1] for p in levels]
    counts = [p.shape[1] * p.shape[2] for p in levels]
    bases = [0, counts[0], counts[0] + counts[1], counts[0] + counts[1] + counts[2]]
    table_rows = sum(counts)

    table = jnp.concatenate([p.reshape(cnt, C) for p, cnt in zip(levels, counts)], axis=0)

    n = boxes.shape[1]
    bx = jnp.transpose(boxes[0], (1, 0))
    bx = jnp.pad(bx, ((0, 0), (0, NB_PAD - n)))

    idx_t, w_t = pl.pallas_call(
        functools.partial(_prelude_body, sizes, bases),
        out_shape=[
            jax.ShapeDtypeStruct((NE_PAD, NB_PAD), jnp.int32),
            jax.ShapeDtypeStruct((NW_PAD, NB_PAD), jnp.float32),
        ],
        in_specs=[
            pl.BlockSpec(memory_space=pltpu.SMEM),
            pl.BlockSpec(memory_space=pltpu.VMEM),
        ],
    )(metadata, bx)

    idx_all = jnp.transpose(idx_t, (1, 0)).reshape(-1)
    w_all = jnp.transpose(w_t, (1, 0)).reshape(-1)

    out = _make_sc_kernel(table_rows, C)(table, idx_all, w_all)
    return out.reshape(1, n, PH, PW, C)


# speedup vs baseline: 5.3912x; 1.1004x over previous
import functools

import numpy as np
import jax
import jax.numpy as jnp
from jax import lax
from jax.experimental import pallas as pl
from jax.experimental.pallas import tpu as pltpu
from jax.experimental.pallas import tpu_sc as plsc

NB = 1000
NB_PAD = 1024
PH = PW = 7
NS = PH * PW
NE = 4 * NS
NE_PAD = 208
NE_G = 200
NW_PAD = 112
CHUNK = 104
SAMP_A = CHUNK // 4
NUM_WORKERS = 32
BOX_PER_W = NB_PAD // NUM_WORKERS

_FY = [float(np.float32(i) / np.float32(6.0)) for i in range(7)]


def _prelude_body(sizes, bases, meta_ref, bx_ref, idx_ref, w_ref):
    rows = meta_ref[0, 0]
    cols = meta_ref[0, 1]
    x1 = bx_ref[0, :]
    y1 = bx_ref[1, :]
    x2 = bx_ref[2, :]
    y2 = bx_ref[3, :]
    h = y2 - y1
    w = x2 - x1
    image_area = rows * cols
    rl = jnp.log(jnp.sqrt(h * w) / jnp.sqrt(image_area)) / jnp.log(2.0)
    rl = jnp.minimum(5.0, jnp.maximum(2.0, 4.0 + jnp.round(rl)))
    lvl = (rl - 2.0).astype(jnp.int32)

    Si = jnp.full(lvl.shape, sizes[0], jnp.int32)
    base = jnp.full(lvl.shape, bases[0], jnp.int32)
    for L in range(1, 4):
        Si = jnp.where(lvl == L, jnp.int32(sizes[L]), Si)
        base = jnp.where(lvl == L, jnp.int32(bases[L]), base)
    Sf = Si.astype(jnp.float32)
    Hm1 = Sf - 1.0

    y1n = y1 / rows
    x1n = x1 / cols
    y2n = y2 / rows
    x2n = x2 / cols

    ygrid = []
    xgrid = []
    for i in range(7):
        ys = (y1n + _FY[i] * (y2n - y1n)) * Hm1
        y0f = jnp.clip(jnp.floor(ys), 0.0, Hm1)
        y0 = y0f.astype(jnp.int32)
        y1c = jnp.clip(y0 + 1, 0, Si - 1)
        wy = jnp.clip(ys - y0f, 0.0, 1.0)
        ygrid.append((y0, y1c, wy))
        xs = (x1n + _FY[i] * (x2n - x1n)) * Hm1
        x0f = jnp.clip(jnp.floor(xs), 0.0, Hm1)
        x0 = x0f.astype(jnp.int32)
        x1c = jnp.clip(x0 + 1, 0, Si - 1)
        wx = jnp.clip(xs - x0f, 0.0, 1.0)
        xgrid.append((x0, x1c, wx))

    for i in range(7):
        y0, y1c, wy = ygrid[i]
        top = base + y0 * Si
        bot = base + y1c * Si
        for j in range(7):
            x0, x1c, wx = xgrid[j]
            s = i * 7 + j
            idx_ref[4 * s + 0, :] = top + x0
            idx_ref[4 * s + 1, :] = top + x1c
            idx_ref[4 * s + 2, :] = bot + x0
            idx_ref[4 * s + 3, :] = bot + x1c
            w_ref[2 * s + 0, :] = wx
            w_ref[2 * s + 1, :] = wy
    idx_ref[pl.ds(NE, NE_PAD - NE), :] = jnp.zeros(
        (NE_PAD - NE, NB_PAD), jnp.int32)
    w_ref[pl.ds(2 * NS, NW_PAD - 2 * NS), :] = jnp.zeros(
        (NW_PAD - 2 * NS, NB_PAD), jnp.float32)


def _make_sc_kernel(table_rows, C):
    mesh = plsc.VectorSubcoreMesh(core_axis_name="c", subcore_axis_name="s")

    @functools.partial(
        pl.kernel,
        mesh=mesh,
        out_type=jax.ShapeDtypeStruct((NB * NS * C,), jnp.float32),
        scratch_types=[
            pltpu.VMEM((BOX_PER_W * NE_PAD,), jnp.int32),
            pltpu.VMEM((BOX_PER_W * NW_PAD,), jnp.float32),
            pltpu.VMEM((NE_G, C), jnp.float32),
            pltpu.VMEM((NE_G, C), jnp.float32),
            pltpu.VMEM((NS * C,), jnp.float32),
            pltpu.SemaphoreType.DMA,
            pltpu.SemaphoreType.DMA,
        ],
    )
    def sc_kernel(table, idx_all, w_all, out, idx_sub, w_sub,
                  rows0, rows1, out_v, sem0, sem1):
        wid = lax.axis_index("s") * 2 + lax.axis_index("c")

        pltpu.sync_copy(
            idx_all.at[pl.ds(wid * BOX_PER_W * NE_PAD, BOX_PER_W * NE_PAD)],
            idx_sub)
        pltpu.sync_copy(
            w_all.at[pl.ds(wid * BOX_PER_W * NW_PAD, BOX_PER_W * NW_PAD)],
            w_sub)

        def issue(t, buf, sem):
            o = t * NE_PAD
            pltpu.async_copy(table.at[idx_sub.at[pl.ds(o, CHUNK)]],
                             buf.at[pl.ds(0, CHUNK)], sem)
            pltpu.async_copy(table.at[idx_sub.at[pl.ds(o + CHUNK, NE_G - CHUNK)]],
                             buf.at[pl.ds(CHUNK, NE_G - CHUNK)], sem)

        def wait_buf(buf, sem):
            pltpu.make_async_copy(table.at[pl.ds(0, CHUNK)],
                                  buf.at[pl.ds(0, CHUNK)], sem).wait()
            pltpu.make_async_copy(table.at[pl.ds(0, NE_G - CHUNK)],
                                  buf.at[pl.ds(CHUNK, NE_G - CHUNK)], sem).wait()

        def compute(t, buf):
            @plsc.parallel_loop(0, NS, unroll=2)
            def _(s):
                wv = w_sub[pl.ds(t * NW_PAD + 2 * s, 16)]
                wx = jnp.full((16,), wv[0], jnp.float32)
                wy = jnp.full((16,), wv[1], jnp.float32)
                owx = 1.0 - wx
                owy = 1.0 - wy
                w00 = owy * owx
                w01 = owy * wx
                w10 = wy * owx
                w11 = wy * wx
                e = 4 * s
                for k in range(C // 16):
                    sl = pl.ds(16 * k, 16)
                    acc = (w00 * buf[e + 0, sl] + w01 * buf[e + 1, sl]
                           + w10 * buf[e + 2, sl] + w11 * buf[e + 3, sl])
                    out_v[pl.ds(s * C + 16 * k, 16)] = acc

        issue(0, rows0, sem0)
        issue(1, rows1, sem1)

        def pair_body(u, _):
            for par, (buf, sem) in enumerate(((rows0, sem0), (rows1, sem1))):
                t = 2 * u + par
                b = wid * BOX_PER_W + t
                wait_buf(buf, sem)
                compute(t, buf)

                @pl.when(b < NB)
                def _():
                    pltpu.sync_copy(out_v, out.at[pl.ds(b * NS * C, NS * C)])

                @pl.when(t + 2 < BOX_PER_W)
                def _():
                    issue(t + 2, buf, sem)

            return 0

        lax.fori_loop(0, BOX_PER_W // 2, pair_body, 0)

    return sc_kernel


def kernel(metadata, boxes, p2, p3, p4, p5):
    C = p2.shape[-1]
    levels = [p2, p3, p4, p5]
    sizes = [p.shape[1] for p in levels]
    counts = [p.shape[1] * p.shape[2] for p in levels]
    bases = [0, counts[0], counts[0] + counts[1], counts[0] + counts[1] + counts[2]]
    table_rows = sum(counts)

    table = jnp.concatenate([p.reshape(cnt, C) for p, cnt in zip(levels, counts)], axis=0)

    n = boxes.shape[1]
    bx = jnp.transpose(boxes[0], (1, 0))
    bx = jnp.pad(bx, ((0, 0), (0, NB_PAD - n)))

    idx_t, w_t = pl.pallas_call(
        functools.partial(_prelude_body, sizes, bases),
        out_shape=[
            jax.ShapeDtypeStruct((NE_PAD, NB_PAD), jnp.int32),
            jax.ShapeDtypeStruct((NW_PAD, NB_PAD), jnp.float32),
        ],
        in_specs=[
            pl.BlockSpec(memory_space=pltpu.SMEM),
            pl.BlockSpec(memory_space=pltpu.VMEM),
        ],
    )(metadata, bx)

    idx_all = jnp.transpose(idx_t, (1, 0)).reshape(-1)
    w_all = jnp.transpose(w_t, (1, 0)).reshape(-1)

    out = _make_sc_kernel(table_rows, C)(table, idx_all, w_all)
    return out.reshape(1, n, PH, PW, C)


# speedup vs baseline: 7.1398x; 1.3243x over previous
import functools

import numpy as np
import jax
import jax.numpy as jnp
from jax import lax
from jax.experimental import pallas as pl
from jax.experimental.pallas import tpu as pltpu
from jax.experimental.pallas import tpu_sc as plsc

NB = 1000
NB_PAD = 1024
PH = PW = 7
NS = PH * PW
NE = 4 * NS
NE_PAD = 208
NE_G = 200
NW_PAD = 112
CHUNK = 104
SAMP_A = CHUNK // 4
NUM_WORKERS = 32
BOX_PER_W = NB_PAD // NUM_WORKERS

_FY = [float(np.float32(i) / np.float32(6.0)) for i in range(7)]


def _prelude_body(sizes, meta_ref, bx_ref, idx_ref, w_ref):
    rows = meta_ref[0, 0]
    cols = meta_ref[0, 1]
    x1 = bx_ref[0, :]
    y1 = bx_ref[1, :]
    x2 = bx_ref[2, :]
    y2 = bx_ref[3, :]
    h = y2 - y1
    w = x2 - x1
    image_area = rows * cols
    rl = jnp.log(jnp.sqrt(h * w) / jnp.sqrt(image_area)) / jnp.log(2.0)
    rl = jnp.minimum(5.0, jnp.maximum(2.0, 4.0 + jnp.round(rl)))
    lvl = (rl - 2.0).astype(jnp.int32)

    Si = jnp.full(lvl.shape, sizes[0], jnp.int32)
    for L in range(1, 4):
        Si = jnp.where(lvl == L, jnp.int32(sizes[L]), Si)
    Sf = Si.astype(jnp.float32)
    Hm1 = Sf - 1.0

    y1n = y1 / rows
    x1n = x1 / cols
    y2n = y2 / rows
    x2n = x2 / cols

    ygrid = []
    xgrid = []
    for i in range(7):
        ys = (y1n + _FY[i] * (y2n - y1n)) * Hm1
        y0f = jnp.clip(jnp.floor(ys), 0.0, Hm1)
        y0 = y0f.astype(jnp.int32)
        y1c = jnp.clip(y0 + 1, 0, Si - 1)
        wy = jnp.clip(ys - y0f, 0.0, 1.0)
        ygrid.append((y0, y1c, wy))
        xs = (x1n + _FY[i] * (x2n - x1n)) * Hm1
        x0f = jnp.clip(jnp.floor(xs), 0.0, Hm1)
        x0 = x0f.astype(jnp.int32)
        x1c = jnp.clip(x0 + 1, 0, Si - 1)
        wx = jnp.clip(xs - x0f, 0.0, 1.0)
        xgrid.append((x0, x1c, wx))

    for i in range(7):
        y0, y1c, wy = ygrid[i]
        top = y0 * Si
        bot = y1c * Si
        for j in range(7):
            x0, x1c, wx = xgrid[j]
            s = i * 7 + j
            idx_ref[4 * s + 0, :] = top + x0
            idx_ref[4 * s + 1, :] = top + x1c
            idx_ref[4 * s + 2, :] = bot + x0
            idx_ref[4 * s + 3, :] = bot + x1c
            w_ref[2 * s + 0, :] = wx
            w_ref[2 * s + 1, :] = wy
    idx_ref[NE, :] = lvl
    idx_ref[pl.ds(NE + 1, NE_PAD - NE - 1), :] = jnp.zeros(
        (NE_PAD - NE - 1, NB_PAD), jnp.int32)
    w_ref[pl.ds(2 * NS, NW_PAD - 2 * NS), :] = jnp.zeros(
        (NW_PAD - 2 * NS, NB_PAD), jnp.float32)


def _make_sc_kernel(C):
    mesh = plsc.VectorSubcoreMesh(core_axis_name="c", subcore_axis_name="s")

    @functools.partial(
        pl.kernel,
        mesh=mesh,
        out_type=jax.ShapeDtypeStruct((NB * NS * C,), jnp.float32),
        scratch_types=[
            pltpu.VMEM((BOX_PER_W * NE_PAD,), jnp.int32),
            pltpu.VMEM((BOX_PER_W * NW_PAD,), jnp.float32),
            pltpu.VMEM((NE_G, C), jnp.float32),
            pltpu.VMEM((NE_G, C), jnp.float32),
            pltpu.VMEM((NS * C,), jnp.float32),
            pltpu.SemaphoreType.DMA,
            pltpu.SemaphoreType.DMA,
        ],
    )
    def sc_kernel(t2, t3, t4, t5, idx_all, w_all, out, idx_sub, w_sub,
                  rows0, rows1, out_v, sem0, sem1):
        wid = lax.axis_index("s") * 2 + lax.axis_index("c")
        tables = (t2, t3, t4, t5)

        pltpu.sync_copy(
            idx_all.at[pl.ds(wid * BOX_PER_W * NE_PAD, BOX_PER_W * NE_PAD)],
            idx_sub)
        pltpu.sync_copy(
            w_all.at[pl.ds(wid * BOX_PER_W * NW_PAD, BOX_PER_W * NW_PAD)],
            w_sub)

        def issue(t, buf, sem):
            o = t * NE_PAD
            lvec = idx_sub[pl.ds(o + NE - 4, 16)]
            lv = lvec[4]
            for L in range(4):
                @pl.when(lv == L)
                def _(tab=tables[L]):
                    pltpu.async_copy(tab.at[idx_sub.at[pl.ds(o, CHUNK)]],
                                     buf.at[pl.ds(0, CHUNK)], sem)
                    pltpu.async_copy(
                        tab.at[idx_sub.at[pl.ds(o + CHUNK, NE_G - CHUNK)]],
                        buf.at[pl.ds(CHUNK, NE_G - CHUNK)], sem)

        def wait_buf(buf, sem):
            pltpu.make_async_copy(t2.at[pl.ds(0, CHUNK)],
                                  buf.at[pl.ds(0, CHUNK)], sem).wait()
            pltpu.make_async_copy(t2.at[pl.ds(0, NE_G - CHUNK)],
                                  buf.at[pl.ds(CHUNK, NE_G - CHUNK)], sem).wait()

        def compute(t, buf):
            @plsc.parallel_loop(0, NS, unroll=2)
            def _(s):
                wv = w_sub[pl.ds(t * NW_PAD + 2 * s, 16)]
                wx = jnp.full((16,), wv[0], jnp.float32)
                wy = jnp.full((16,), wv[1], jnp.float32)
                owx = 1.0 - wx
                owy = 1.0 - wy
                w00 = owy * owx
                w01 = owy * wx
                w10 = wy * owx
                w11 = wy * wx
                e = 4 * s
                for k in range(C // 16):
                    sl = pl.ds(16 * k, 16)
                    acc = (w00 * buf[e + 0, sl] + w01 * buf[e + 1, sl]
                           + w10 * buf[e + 2, sl] + w11 * buf[e + 3, sl])
                    out_v[pl.ds(s * C + 16 * k, 16)] = acc

        issue(0, rows0, sem0)
        issue(1, rows1, sem1)

        def pair_body(u, _):
            for par, (buf, sem) in enumerate(((rows0, sem0), (rows1, sem1))):
                t = 2 * u + par
                b = wid * BOX_PER_W + t
                wait_buf(buf, sem)
                compute(t, buf)

                @pl.when(b < NB)
                def _():
                    pltpu.sync_copy(out_v, out.at[pl.ds(b * NS * C, NS * C)])

                @pl.when(t + 2 < BOX_PER_W)
                def _():
                    issue(t + 2, buf, sem)

            return 0

        lax.fori_loop(0, BOX_PER_W // 2, pair_body, 0)

    return sc_kernel


def kernel(metadata, boxes, p2, p3, p4, p5):
    C = p2.shape[-1]
    levels = [p2, p3, p4, p5]
    sizes = [p.shape[1] for p in levels]
    counts = [p.shape[1] * p.shape[2] for p in levels]
    tabs = [p.reshape(cnt, C) for p, cnt in zip(levels, counts)]

    n = boxes.shape[1]
    bx = jnp.transpose(boxes[0], (1, 0))
    bx = jnp.pad(bx, ((0, 0), (0, NB_PAD - n)))

    idx_t, w_t = pl.pallas_call(
        functools.partial(_prelude_body, sizes),
        out_shape=[
            jax.ShapeDtypeStruct((NE_PAD, NB_PAD), jnp.int32),
            jax.ShapeDtypeStruct((NW_PAD, NB_PAD), jnp.float32),
        ],
        in_specs=[
            pl.BlockSpec(memory_space=pltpu.SMEM),
            pl.BlockSpec(memory_space=pltpu.VMEM),
        ],
    )(metadata, bx)

    idx_all = jnp.transpose(idx_t, (1, 0)).reshape(-1)
    w_all = jnp.transpose(w_t, (1, 0)).reshape(-1)

    out = _make_sc_kernel(C)(*tabs, idx_all, w_all)
    return out.reshape(1, n, PH, PW, C)
